# Initial kernel scaffold; baseline (speedup 1.0000x reference)
import jax
import jax.numpy as jnp
from jax.experimental import pallas as pl


def kernel(x0, edge_index0, edge_attr, batch, W_msg, W_edge, W_root, b, W1, b1, W2, b2):
    raise NotImplementedError("write your pallas kernel here")


# speedup vs baseline: 3.1488x; 3.1488x over previous
import functools

import jax
import jax.numpy as jnp
from jax import lax
from jax.experimental import pallas as pl
from jax.experimental.pallas import tpu as pltpu
from jax.experimental.pallas import tpu_sc as plsc


def _pick_chunk(epw: int) -> int:
    for c in range(128, 7, -8):
        if epw % c == 0:
            return c
    raise ValueError(f"no valid edge chunk for {epw} edges/worker")


def _sc_segsum_128(dst, N, *, table=None, src=None, rows_hbm=None):
    E = dst.shape[0]
    D = 128
    gather = table is not None

    info = plsc.get_sparse_core_info()
    NC, NS = info.num_cores, info.num_subcores
    NW = NC * NS
    assert E % NW == 0
    epw = E // NW
    CH = _pick_chunk(epw)
    n_chunks = epw // CH
    n_zchunks = -(-N // (NS * CH))
    rows_per_tile = n_zchunks * CH
    Npad = NS * rows_per_tile

    mesh = plsc.VectorSubcoreMesh(core_axis_name="c", subcore_axis_name="s")

    def body(dst_hbm, src_hbm, tab_hbm, lin_hbm, zx_hbm, out,
             acc, idx_s, idx_d, rows, sem, idx_z):
        cid = lax.axis_index("c")
        sid = lax.axis_index("s")
        wid = sid * NC + cid

        iota16 = lax.broadcasted_iota(jnp.int32, (16,), 0)
        tile_base = sid * rows_per_tile
        for kk in range(n_zchunks):
            for t in range(CH // 16):
                idx_z[kk][pl.ds(t * 16, 16)] = (
                    iota16 + (tile_base + kk * CH + t * 16))

        pltpu.sync_copy(zx_hbm, rows)
        for kk in range(n_zchunks):
            pltpu.sync_copy(rows, acc.at[idx_z[kk]])
        plsc.subcore_barrier()

        def edge_step(j, carry):
            base = wid * epw + j * CH
            pltpu.sync_copy(dst_hbm.at[pl.ds(base, CH)], idx_d)
            if gather:
                pltpu.sync_copy(src_hbm.at[pl.ds(base, CH)], idx_s)
                pltpu.async_copy(tab_hbm.at[idx_s], rows, sem).wait()
            else:
                pltpu.sync_copy(lin_hbm.at[pl.ds(base, CH)], rows)
            pltpu.sync_copy(rows, acc.at[idx_d], add=True)
            return carry

        lax.fori_loop(0, n_chunks, edge_step, 0)
        plsc.subcore_barrier()

        out_base = cid * Npad + tile_base
        for kk in range(n_zchunks):
            pltpu.async_copy(acc.at[idx_z[kk]], rows, sem).wait()
            pltpu.sync_copy(rows, out.at[pl.ds(out_base + kk * CH, CH)])

    scratch = [
        pltpu.VMEM_SHARED((Npad, D), jnp.float32),
        pltpu.VMEM((CH,), jnp.int32),
        pltpu.VMEM((CH,), jnp.int32),
        pltpu.VMEM((CH, D), jnp.float32),
        pltpu.SemaphoreType.DMA,
    ] + [pltpu.VMEM((CH,), jnp.int32) for _ in range(n_zchunks)]
    out_t = jax.ShapeDtypeStruct((NC * Npad, D), jnp.float32)
    zx = jnp.zeros((CH, D), jnp.float32)

    if gather:
        @functools.partial(pl.kernel, out_type=out_t, mesh=mesh,
                           scratch_types=scratch)
        def kg(dst_hbm, src_hbm, tab_hbm, zx_hbm, out,
               acc, idx_s, idx_d, rows, sem, *idx_z):
            body(dst_hbm, src_hbm, tab_hbm, None, zx_hbm, out,
                 acc, idx_s, idx_d, rows, sem, idx_z)

        out2 = kg(dst, src, table, zx)
    else:
        @functools.partial(pl.kernel, out_type=out_t, mesh=mesh,
                           scratch_types=scratch)
        def kl(dst_hbm, lin_hbm, zx_hbm, out,
               acc, idx_s, idx_d, rows, sem, *idx_z):
            body(dst_hbm, None, None, lin_hbm, zx_hbm, out,
                 acc, idx_s, idx_d, rows, sem, idx_z)

        out2 = kl(dst, rows_hbm, zx)
    return out2.reshape(NC, Npad, D)


def _tc_dense(part_x, part_e, x0, batch3, W_msg, W_edge, W_root, b2d,
              W1, b1_2d, W2, b2_2d, Bn):
    NC, Npad, D = part_x.shape
    DE = part_e.shape[2]
    N = x0.shape[0]
    H1 = W1.shape[1]
    G = 128
    OUT = W2.shape[1]
    nblk = N // Bn

    def body(px, pe, xb, bb, Wm, We, Wr, bv, W1r, b1r, W2r, b2r,
             out, pooled, cnt):
        i = pl.program_id(0)

        @pl.when(i == 0)
        def _init():
            pooled[...] = jnp.zeros_like(pooled)
            cnt[...] = jnp.zeros_like(cnt)

        agg_x = px[0] + px[1]
        agg_e = pe[0] + pe[1]
        h = agg_x @ Wm[...] + agg_e @ We[...] + xb[...] @ Wr[...] + bv[...]
        h = jnp.maximum(h, 0.0)
        h2 = jnp.maximum(h @ W1r[...] + b1r[...], 0.0)

        seg = bb[0, 0, :]
        oh = (seg[:, None] == lax.broadcasted_iota(jnp.int32, (Bn, G), 1))
        oh = oh.astype(jnp.float32)
        pooled[...] += lax.dot_general(oh, h2, (((0,), (0,)), ((), ())))
        cnt[...] += lax.dot_general(
            oh, jnp.ones((Bn, 8), jnp.float32), (((0,), (0,)), ((), ())))

        @pl.when(i == nblk - 1)
        def _final():
            denom = jnp.maximum(cnt[:, 0:1], 1.0)
            pm = pooled[...] / denom
            out[...] = pm @ W2r[...] + b2r[...]

    return pl.pallas_call(
        body,
        grid=(nblk,),
        in_specs=[
            pl.BlockSpec((NC, Bn, D), lambda i: (0, i, 0)),
            pl.BlockSpec((NC, Bn, DE), lambda i: (0, i, 0)),
            pl.BlockSpec((Bn, D), lambda i: (i, 0)),
            pl.BlockSpec((1, 1, Bn), lambda i: (i, 0, 0)),
            pl.BlockSpec((D, D), lambda i: (0, 0)),
            pl.BlockSpec((DE, D), lambda i: (0, 0)),
            pl.BlockSpec((D, D), lambda i: (0, 0)),
            pl.BlockSpec((1, D), lambda i: (0, 0)),
            pl.BlockSpec((D, H1), lambda i: (0, 0)),
            pl.BlockSpec((1, H1), lambda i: (0, 0)),
            pl.BlockSpec((H1, OUT), lambda i: (0, 0)),
            pl.BlockSpec((1, OUT), lambda i: (0, 0)),
        ],
        out_specs=pl.BlockSpec((G, OUT), lambda i: (0, 0)),
        out_shape=jax.ShapeDtypeStruct((G, OUT), jnp.float32),
        scratch_shapes=[
            pltpu.VMEM((G, H1), jnp.float32),
            pltpu.VMEM((G, 8), jnp.float32),
        ],
    )(part_x, part_e, x0, batch3, W_msg, W_edge, W_root, b2d,
      W1, b1_2d, W2, b2_2d)


def kernel(x0, edge_index0, edge_attr, batch, W_msg, W_edge, W_root, b,
           W1, b1, W2, b2):
    N, D = x0.shape
    DE = edge_attr.shape[1]
    src = edge_index0[0]
    dst = edge_index0[1]

    ea128 = jnp.pad(edge_attr, ((0, 0), (0, 128 - DE)))
    We128 = jnp.pad(W_edge, ((0, 128 - DE), (0, 0)))
    part_x = _sc_segsum_128(dst, N, table=x0, src=src)
    part_e = _sc_segsum_128(dst, N, rows_hbm=ea128)

    Bn = 1000 if N % 1000 == 0 else 8
    batch3 = batch.reshape(N // Bn, 1, Bn)
    out = _tc_dense(
        part_x, part_e, x0, batch3, W_msg, We128, W_root,
        b.reshape(1, -1), W1, b1.reshape(1, -1), W2, b2.reshape(1, -1), Bn)
    return out.reshape(-1)


# speedup vs baseline: 3.6374x; 1.1552x over previous
import functools

import jax
import jax.numpy as jnp
from jax import lax
from jax.experimental import pallas as pl
from jax.experimental.pallas import tpu as pltpu
from jax.experimental.pallas import tpu_sc as plsc


def _pick_chunk(epw: int) -> int:
    for c in range(128, 7, -8):
        if epw % c == 0:
            return c
    raise ValueError(f"no valid edge chunk for {epw} edges/worker")


def _sc_segsum_128(dst, N, *, table=None, src=None, rows_hbm=None):
    E = dst.shape[0]
    D = 128
    gather = table is not None

    info = plsc.get_sparse_core_info()
    NC, NS = info.num_cores, info.num_subcores
    NW = NC * NS
    assert E % NW == 0
    epw = E // NW
    CH = _pick_chunk(epw)
    n_chunks = epw // CH
    n_zchunks = -(-N // (NS * CH))
    rows_per_tile = n_zchunks * CH
    Npad = NS * rows_per_tile

    mesh = plsc.VectorSubcoreMesh(core_axis_name="c", subcore_axis_name="s")

    def body(dst_hbm, src_hbm, tab_hbm, lin_hbm, zx_hbm, out,
             acc, idx_s, idx_d, rows, sem, idx_z, pk=None):
        cid = lax.axis_index("c")
        sid = lax.axis_index("s")
        wid = sid * NC + cid

        iota16 = lax.broadcasted_iota(jnp.int32, (16,), 0)
        tile_base = sid * rows_per_tile
        for kk in range(n_zchunks):
            for t in range(CH // 16):
                idx_z[kk][pl.ds(t * 16, 16)] = (
                    iota16 + (tile_base + kk * CH + t * 16))

        pltpu.sync_copy(zx_hbm, rows)
        for kk in range(n_zchunks):
            pltpu.sync_copy(rows, acc.at[idx_z[kk]])
        plsc.subcore_barrier()

        def edge_step(j, carry):
            base = wid * epw + j * CH
            pltpu.sync_copy(dst_hbm.at[pl.ds(base, CH)], idx_d)
            if gather:
                pltpu.sync_copy(src_hbm.at[pl.ds(base, CH)], idx_s)
                pltpu.async_copy(tab_hbm.at[idx_s], rows, sem).wait()
            else:
                pltpu.sync_copy(lin_hbm.at[pl.ds(base * 16, CH * 16)], pk)
                for r in range(CH):
                    rows[r, pl.ds(0, 16)] = pk[pl.ds(r * 16, 16)]
            pltpu.sync_copy(rows, acc.at[idx_d], add=True)
            return carry

        lax.fori_loop(0, n_chunks, edge_step, 0)
        plsc.subcore_barrier()

        out_base = cid * Npad + tile_base
        for kk in range(n_zchunks):
            pltpu.async_copy(acc.at[idx_z[kk]], rows, sem).wait()
            pltpu.sync_copy(rows, out.at[pl.ds(out_base + kk * CH, CH)])

    scratch = [
        pltpu.VMEM_SHARED((Npad, D), jnp.float32),
        pltpu.VMEM((CH,), jnp.int32),
        pltpu.VMEM((CH,), jnp.int32),
        pltpu.VMEM((CH, D), jnp.float32),
        pltpu.SemaphoreType.DMA,
    ] + [pltpu.VMEM((CH,), jnp.int32) for _ in range(n_zchunks)]
    out_t = jax.ShapeDtypeStruct((NC * Npad, D), jnp.float32)
    zx = jnp.zeros((CH, D), jnp.float32)

    if gather:
        @functools.partial(pl.kernel, out_type=out_t, mesh=mesh,
                           scratch_types=scratch)
        def kg(dst_hbm, src_hbm, tab_hbm, zx_hbm, out,
               acc, idx_s, idx_d, rows, sem, *idx_z):
            body(dst_hbm, src_hbm, tab_hbm, None, zx_hbm, out,
                 acc, idx_s, idx_d, rows, sem, idx_z)

        out2 = kg(dst, src, table, zx)
    else:
        @functools.partial(pl.kernel, out_type=out_t, mesh=mesh,
                           scratch_types=scratch
                           + [pltpu.VMEM((CH * 16,), jnp.float32)])
        def kl(dst_hbm, lin_hbm, zx_hbm, out,
               acc, idx_s, idx_d, rows, sem, *idx_zp):
            body(dst_hbm, None, None, lin_hbm, zx_hbm, out,
                 acc, idx_s, idx_d, rows, sem, idx_zp[:-1], pk=idx_zp[-1])

        out2 = kl(dst, rows_hbm, zx)
    return out2.reshape(NC, Npad, D)


def _tc_dense(part_x, part_e, x0, batch3, W_msg, W_edge, W_root, b2d,
              W1, b1_2d, W2, b2_2d, Bn):
    NC, Npad, D = part_x.shape
    DE = part_e.shape[2]
    N = x0.shape[0]
    H1 = W1.shape[1]
    G = 128
    OUT = W2.shape[1]
    nblk = N // Bn

    def body(px, pe, xb, bb, Wm, We, Wr, bv, W1r, b1r, W2r, b2r,
             out, pooled, cnt):
        i = pl.program_id(0)

        @pl.when(i == 0)
        def _init():
            pooled[...] = jnp.zeros_like(pooled)
            cnt[...] = jnp.zeros_like(cnt)

        agg_x = px[0] + px[1]
        agg_e = pe[0] + pe[1]
        h = agg_x @ Wm[...] + agg_e @ We[...] + xb[...] @ Wr[...] + bv[...]
        h = jnp.maximum(h, 0.0)
        h2 = jnp.maximum(h @ W1r[...] + b1r[...], 0.0)

        seg = bb[0, 0, :]
        oh = (seg[:, None] == lax.broadcasted_iota(jnp.int32, (Bn, G), 1))
        oh = oh.astype(jnp.float32)
        pooled[...] += lax.dot_general(oh, h2, (((0,), (0,)), ((), ())))
        cnt[...] += lax.dot_general(
            oh, jnp.ones((Bn, 8), jnp.float32), (((0,), (0,)), ((), ())))

        @pl.when(i == nblk - 1)
        def _final():
            denom = jnp.maximum(cnt[:, 0:1], 1.0)
            pm = pooled[...] / denom
            out[...] = pm @ W2r[...] + b2r[...]

    return pl.pallas_call(
        body,
        grid=(nblk,),
        in_specs=[
            pl.BlockSpec((NC, Bn, D), lambda i: (0, i, 0)),
            pl.BlockSpec((NC, Bn, DE), lambda i: (0, i, 0)),
            pl.BlockSpec((Bn, D), lambda i: (i, 0)),
            pl.BlockSpec((1, 1, Bn), lambda i: (i, 0, 0)),
            pl.BlockSpec((D, D), lambda i: (0, 0)),
            pl.BlockSpec((DE, D), lambda i: (0, 0)),
            pl.BlockSpec((D, D), lambda i: (0, 0)),
            pl.BlockSpec((1, D), lambda i: (0, 0)),
            pl.BlockSpec((D, H1), lambda i: (0, 0)),
            pl.BlockSpec((1, H1), lambda i: (0, 0)),
            pl.BlockSpec((H1, OUT), lambda i: (0, 0)),
            pl.BlockSpec((1, OUT), lambda i: (0, 0)),
        ],
        out_specs=pl.BlockSpec((G, OUT), lambda i: (0, 0)),
        out_shape=jax.ShapeDtypeStruct((G, OUT), jnp.float32),
        scratch_shapes=[
            pltpu.VMEM((G, H1), jnp.float32),
            pltpu.VMEM((G, 8), jnp.float32),
        ],
    )(part_x, part_e, x0, batch3, W_msg, W_edge, W_root, b2d,
      W1, b1_2d, W2, b2_2d)


def kernel(x0, edge_index0, edge_attr, batch, W_msg, W_edge, W_root, b,
           W1, b1, W2, b2):
    N, D = x0.shape
    DE = edge_attr.shape[1]
    src = edge_index0[0]
    dst = edge_index0[1]

    assert DE == 16
    eap = edge_attr.reshape(-1)
    We128 = jnp.pad(W_edge, ((0, 128 - DE), (0, 0)))
    part_x = _sc_segsum_128(dst, N, table=x0, src=src)
    part_e = _sc_segsum_128(dst, N, rows_hbm=eap)

    Bn = 1000 if N % 1000 == 0 else 8
    batch3 = batch.reshape(N // Bn, 1, Bn)
    out = _tc_dense(
        part_x, part_e, x0, batch3, W_msg, We128, W_root,
        b.reshape(1, -1), W1, b1.reshape(1, -1), W2, b2.reshape(1, -1), Bn)
    return out.reshape(-1)


# speedup vs baseline: 4.7994x; 1.3195x over previous
import functools

import jax
import jax.numpy as jnp
from jax import lax
from jax.experimental import pallas as pl
from jax.experimental.pallas import tpu as pltpu
from jax.experimental.pallas import tpu_sc as plsc


def _pick_chunk(epw: int) -> int:
    for c in range(128, 7, -8):
        if epw % c == 0:
            return c
    raise ValueError(f"no valid edge chunk for {epw} edges/worker")


def _sc_segsum_128(dst, N, *, table=None, src=None, rows_hbm=None):
    E = dst.shape[0]
    D = 128
    gather = table is not None

    info = plsc.get_sparse_core_info()
    NC, NS = info.num_cores, info.num_subcores
    NW = NC * NS
    assert E % NW == 0
    epw = E // NW
    CH = _pick_chunk(epw)
    n_chunks = epw // CH
    n_zchunks = -(-N // (NS * CH))
    rows_per_tile = n_zchunks * CH
    Npad = NS * rows_per_tile

    mesh = plsc.VectorSubcoreMesh(core_axis_name="c", subcore_axis_name="s")

    n_pairs = n_chunks // 2
    tail = n_chunks - 2 * n_pairs

    def prologue(sid, rows, zx_hbm, acc, idx_z):
        iota16 = lax.broadcasted_iota(jnp.int32, (16,), 0)
        tile_base = sid * rows_per_tile
        for kk in range(n_zchunks):
            for t in range(CH // 16):
                idx_z[kk][pl.ds(t * 16, 16)] = (
                    iota16 + (tile_base + kk * CH + t * 16))
        pltpu.sync_copy(zx_hbm, rows)
        for kk in range(n_zchunks):
            pltpu.sync_copy(rows, acc.at[idx_z[kk]])
        return tile_base

    def writeback(cid, tile_base, rows, sem, acc, out, idx_z):
        out_base = cid * Npad + tile_base
        for kk in range(n_zchunks):
            pltpu.async_copy(acc.at[idx_z[kk]], rows, sem).wait()
            pltpu.sync_copy(rows, out.at[pl.ds(out_base + kk * CH, CH)])

    scratch = [
        pltpu.VMEM_SHARED((Npad, D), jnp.float32),
        pltpu.VMEM((CH,), jnp.int32),
        pltpu.VMEM((CH,), jnp.int32),
        pltpu.VMEM((CH,), jnp.int32),
        pltpu.VMEM((CH,), jnp.int32),
        pltpu.VMEM((CH, D), jnp.float32),
        pltpu.VMEM((CH, D), jnp.float32),
        pltpu.SemaphoreType.DMA,
        pltpu.SemaphoreType.DMA,
    ] + [pltpu.VMEM((CH,), jnp.int32) for _ in range(n_zchunks)]
    out_t = jax.ShapeDtypeStruct((NC * Npad, D), jnp.float32)
    zx = jnp.zeros((CH, D), jnp.float32)

    if gather:
        @functools.partial(pl.kernel, out_type=out_t, mesh=mesh,
                           scratch_types=scratch)
        def kg(dst_hbm, src_hbm, tab_hbm, zx_hbm, out, acc,
               idx_sA, idx_dA, idx_sB, idx_dB, rowsA, rowsB,
               semA, semB, *idx_z):
            cid = lax.axis_index("c")
            sid = lax.axis_index("s")
            wid = sid * NC + cid
            tile_base = prologue(sid, rowsA, zx_hbm, acc, idx_z)
            plsc.subcore_barrier()

            def pair_step(p, carry):
                ba = wid * epw + (2 * p) * CH
                bb = ba + CH
                pltpu.sync_copy(dst_hbm.at[pl.ds(ba, CH)], idx_dA)
                pltpu.sync_copy(src_hbm.at[pl.ds(ba, CH)], idx_sA)
                gA = pltpu.async_copy(tab_hbm.at[idx_sA], rowsA, semA)
                pltpu.sync_copy(dst_hbm.at[pl.ds(bb, CH)], idx_dB)
                pltpu.sync_copy(src_hbm.at[pl.ds(bb, CH)], idx_sB)
                gB = pltpu.async_copy(tab_hbm.at[idx_sB], rowsB, semB)
                gA.wait()
                pltpu.sync_copy(rowsA, acc.at[idx_dA], add=True)
                gB.wait()
                pltpu.sync_copy(rowsB, acc.at[idx_dB], add=True)
                return carry

            lax.fori_loop(0, n_pairs, pair_step, 0)
            for t in range(tail):
                bt = wid * epw + (2 * n_pairs + t) * CH
                pltpu.sync_copy(dst_hbm.at[pl.ds(bt, CH)], idx_dA)
                pltpu.sync_copy(src_hbm.at[pl.ds(bt, CH)], idx_sA)
                pltpu.async_copy(tab_hbm.at[idx_sA], rowsA, semA).wait()
                pltpu.sync_copy(rowsA, acc.at[idx_dA], add=True)
            plsc.subcore_barrier()
            writeback(cid, tile_base, rowsA, semA, acc, out, idx_z)

        out2 = kg(dst, src, table, zx)
    else:
        @functools.partial(pl.kernel, out_type=out_t, mesh=mesh,
                           scratch_types=scratch
                           + [pltpu.VMEM((CH * 16,), jnp.float32),
                              pltpu.VMEM((CH * 16,), jnp.float32)])
        def kl(dst_hbm, lin_hbm, zx_hbm, out, acc,
               idx_sA, idx_dA, idx_sB, idx_dB, rowsA, rowsB,
               semA, semB, *rest):
            idx_z = rest[:n_zchunks]
            pkA, pkB = rest[n_zchunks], rest[n_zchunks + 1]
            cid = lax.axis_index("c")
            sid = lax.axis_index("s")
            wid = sid * NC + cid
            tile_base = prologue(sid, rowsA, zx_hbm, acc, idx_z)
            pltpu.sync_copy(zx_hbm, rowsB)
            plsc.subcore_barrier()

            def unpack(pk, rows):
                for r in range(CH):
                    rows[r, pl.ds(0, 16)] = pk[pl.ds(r * 16, 16)]

            def pair_step(p, carry):
                ba = wid * epw + (2 * p) * CH
                bb = ba + CH
                pltpu.sync_copy(dst_hbm.at[pl.ds(ba, CH)], idx_dA)
                fA = pltpu.async_copy(
                    lin_hbm.at[pl.ds(ba * 16, CH * 16)], pkA, semA)
                pltpu.sync_copy(dst_hbm.at[pl.ds(bb, CH)], idx_dB)
                fB = pltpu.async_copy(
                    lin_hbm.at[pl.ds(bb * 16, CH * 16)], pkB, semB)
                fA.wait()
                unpack(pkA, rowsA)
                pltpu.sync_copy(rowsA, acc.at[idx_dA], add=True)
                fB.wait()
                unpack(pkB, rowsB)
                pltpu.sync_copy(rowsB, acc.at[idx_dB], add=True)
                return carry

            lax.fori_loop(0, n_pairs, pair_step, 0)
            for t in range(tail):
                bt = wid * epw + (2 * n_pairs + t) * CH
                pltpu.sync_copy(dst_hbm.at[pl.ds(bt, CH)], idx_dA)
                pltpu.sync_copy(lin_hbm.at[pl.ds(bt * 16, CH * 16)], pkA)
                unpack(pkA, rowsA)
                pltpu.sync_copy(rowsA, acc.at[idx_dA], add=True)
            plsc.subcore_barrier()
            writeback(cid, tile_base, rowsA, semA, acc, out, idx_z)

        out2 = kl(dst, rows_hbm, zx)
    return out2.reshape(NC, Npad, D)


def _tc_dense(part_x, part_e, x0, batch3, W_msg, W_edge, W_root, b2d,
              W1, b1_2d, W2, b2_2d, Bn):
    NC, Npad, D = part_x.shape
    DE = part_e.shape[2]
    N = x0.shape[0]
    H1 = W1.shape[1]
    G = 128
    OUT = W2.shape[1]
    nblk = N // Bn

    def body(px, pe, xb, bb, Wm, We, Wr, bv, W1r, b1r, W2r, b2r,
             out, pooled, cnt):
        i = pl.program_id(0)

        @pl.when(i == 0)
        def _init():
            pooled[...] = jnp.zeros_like(pooled)
            cnt[...] = jnp.zeros_like(cnt)

        agg_x = px[0] + px[1]
        agg_e = pe[0] + pe[1]
        h = agg_x @ Wm[...] + agg_e @ We[...] + xb[...] @ Wr[...] + bv[...]
        h = jnp.maximum(h, 0.0)
        h2 = jnp.maximum(h @ W1r[...] + b1r[...], 0.0)

        seg = bb[0, 0, :]
        oh = (seg[:, None] == lax.broadcasted_iota(jnp.int32, (Bn, G), 1))
        oh = oh.astype(jnp.float32)
        pooled[...] += lax.dot_general(oh, h2, (((0,), (0,)), ((), ())))
        cnt[...] += lax.dot_general(
            oh, jnp.ones((Bn, 8), jnp.float32), (((0,), (0,)), ((), ())))

        @pl.when(i == nblk - 1)
        def _final():
            denom = jnp.maximum(cnt[:, 0:1], 1.0)
            pm = pooled[...] / denom
            out[...] = pm @ W2r[...] + b2r[...]

    return pl.pallas_call(
        body,
        grid=(nblk,),
        in_specs=[
            pl.BlockSpec((NC, Bn, D), lambda i: (0, i, 0)),
            pl.BlockSpec((NC, Bn, DE), lambda i: (0, i, 0)),
            pl.BlockSpec((Bn, D), lambda i: (i, 0)),
            pl.BlockSpec((1, 1, Bn), lambda i: (i, 0, 0)),
            pl.BlockSpec((D, D), lambda i: (0, 0)),
            pl.BlockSpec((DE, D), lambda i: (0, 0)),
            pl.BlockSpec((D, D), lambda i: (0, 0)),
            pl.BlockSpec((1, D), lambda i: (0, 0)),
            pl.BlockSpec((D, H1), lambda i: (0, 0)),
            pl.BlockSpec((1, H1), lambda i: (0, 0)),
            pl.BlockSpec((H1, OUT), lambda i: (0, 0)),
            pl.BlockSpec((1, OUT), lambda i: (0, 0)),
        ],
        out_specs=pl.BlockSpec((G, OUT), lambda i: (0, 0)),
        out_shape=jax.ShapeDtypeStruct((G, OUT), jnp.float32),
        scratch_shapes=[
            pltpu.VMEM((G, H1), jnp.float32),
            pltpu.VMEM((G, 8), jnp.float32),
        ],
    )(part_x, part_e, x0, batch3, W_msg, W_edge, W_root, b2d,
      W1, b1_2d, W2, b2_2d)


def kernel(x0, edge_index0, edge_attr, batch, W_msg, W_edge, W_root, b,
           W1, b1, W2, b2):
    N, D = x0.shape
    DE = edge_attr.shape[1]
    src = edge_index0[0]
    dst = edge_index0[1]

    assert DE == 16
    eap = edge_attr.reshape(-1)
    We128 = jnp.pad(W_edge, ((0, 128 - DE), (0, 0)))
    part_x = _sc_segsum_128(dst, N, table=x0, src=src)
    part_e = _sc_segsum_128(dst, N, rows_hbm=eap)

    Bn = 1000 if N % 1000 == 0 else 8
    batch3 = batch.reshape(N // Bn, 1, Bn)
    out = _tc_dense(
        part_x, part_e, x0, batch3, W_msg, We128, W_root,
        b.reshape(1, -1), W1, b1.reshape(1, -1), W2, b2.reshape(1, -1), Bn)
    return out.reshape(-1)
